# baseline (speedup 1.0000x reference)
import functools
import math

import jax
import jax.numpy as jnp
from jax import lax
from jax.experimental import pallas as pl
from jax.experimental.pallas import tpu as pltpu
from jax.experimental.pallas import tpu_sc as plsc

DT = 0.1
EPS = 1e-8
C4 = 4.0 * math.log(2.0 * math.pi)
NW = 32
LANES = 16
KEY_MASK = -64
KEY_BIG = 0x7E000000


def _nsqrt(x):
    i = lax.bitcast_convert_type(x, jnp.int32)
    i = jnp.int32(0x5F3759DF) - lax.shift_right_logical(i, 1)
    y = lax.bitcast_convert_type(i, jnp.float32)
    y = y * (1.5 - 0.5 * x * y * y)
    y = y * (1.5 - 0.5 * x * y * y)
    y = y * (1.5 - 0.5 * x * y * y)
    return x * y


def _make_sc_kernel(B, A, NT, M, L, LP):
    NSEG = L - 1
    SEGP = L
    PAIRS = B * A
    CHUNKS = NT // LANES
    ITEMS = PAIRS * CHUNKS
    PER_W = ITEMS // NW
    PTS_W = PER_W * LANES
    PIDX_PAD = 32
    POFF = B * NT
    TOT = B * A * NT
    SEGT = PAIRS * SEGP
    mesh = plsc.VectorSubcoreMesh(core_axis_name="c", subcore_axis_name="s")

    @functools.partial(
        pl.kernel,
        mesh=mesh,
        compiler_params=pltpu.CompilerParams(needs_layout_passes=False),
        out_type=jax.ShapeDtypeStruct((TOT,), jnp.float32),
        scratch_types=[
            pltpu.VMEM((PIDX_PAD,), jnp.int32),
            pltpu.VMEM((PIDX_PAD,), jnp.int32),
            pltpu.VMEM((PIDX_PAD, LP), jnp.float32),
            pltpu.VMEM((PIDX_PAD, LP), jnp.float32),
            pltpu.VMEM((SEGT,), jnp.float32),
            pltpu.VMEM((SEGT,), jnp.float32),
            pltpu.VMEM((SEGT,), jnp.float32),
            pltpu.VMEM((SEGT,), jnp.float32),
            pltpu.VMEM((SEGT,), jnp.float32),
            pltpu.VMEM((SEGT,), jnp.float32),
            pltpu.VMEM((SEGT,), jnp.float32),
            pltpu.VMEM((SEGT,), jnp.float32),
            pltpu.VMEM((6 * POFF,), jnp.float32),
            pltpu.VMEM((PTS_W,), jnp.float32),
            pltpu.VMEM((PTS_W,), jnp.float32),
            pltpu.VMEM((PTS_W,), jnp.float32),
            pltpu.VMEM((64,), jnp.float32),
            pltpu.VMEM((PTS_W,), jnp.float32),
            pltpu.SemaphoreType.DMA,
            pltpu.SemaphoreType.DMA,
        ],
    )
    def sc_kernel(map_x_hbm, map_y_hbm, pidx_hbm, pts_hbm, aux_hbm,
                  consts_hbm, out_hbm,
                  pidx_v, gidx_v, rows_x, rows_y,
                  sp0x, sp0y, svx, svy, siv2, slen, scum, sil,
                  pts_v, gap_v, ttc_v, feas_v, consts_v,
                  out_v, sem_rows, sem_bulk):
        wid = lax.axis_index("s") * 2 + lax.axis_index("c")
        wbase = wid * PTS_W

        cp_pts = pltpu.async_copy(pts_hbm, pts_v, sem_bulk)
        cp_gap = pltpu.async_copy(aux_hbm.at[pl.ds(wbase, PTS_W)], gap_v, sem_bulk)
        cp_ttc = pltpu.async_copy(aux_hbm.at[pl.ds(TOT + wbase, PTS_W)], ttc_v, sem_bulk)
        cp_feas = pltpu.async_copy(aux_hbm.at[pl.ds(2 * TOT + wbase, PTS_W)], feas_v, sem_bulk)
        cp_const = pltpu.async_copy(consts_hbm, consts_v, sem_bulk)

        pltpu.sync_copy(pidx_hbm, pidx_v)
        for c in range(PIDX_PAD // LANES):
            pr = lax.iota(jnp.int32, LANES) + (c * LANES)
            row = pidx_v[pl.ds(c * LANES, LANES)] + (pr // A) * M
            gidx_v[pl.ds(c * LANES, LANES)] = jnp.minimum(row, B * M - 1)
        cp_rx = pltpu.async_copy(map_x_hbm.at[gidx_v], rows_x, sem_rows)
        cp_ry = pltpu.async_copy(map_y_hbm.at[gidx_v], rows_y, sem_rows)
        cp_rx.wait()
        cp_ry.wait()

        def prep_pair(p, carry):
            off = jnp.float32(0.0)
            for c in range(SEGP // LANES):
                x_lo = rows_x[p, pl.ds(c * LANES, LANES)]
                x_hi = rows_x[p, pl.ds(c * LANES + 1, LANES)]
                y_lo = rows_y[p, pl.ds(c * LANES, LANES)]
                y_hi = rows_y[p, pl.ds(c * LANES + 1, LANES)]
                vx_ = x_hi - x_lo
                vy_ = y_hi - y_lo
                v2 = jnp.maximum(vx_ * vx_ + vy_ * vy_, EPS)
                ln = _nsqrt(v2)
                cs = plsc.cumsum(ln)
                base = p * SEGP + c * LANES
                sp0x[pl.ds(base, LANES)] = x_lo
                sp0y[pl.ds(base, LANES)] = y_lo
                svx[pl.ds(base, LANES)] = vx_
                svy[pl.ds(base, LANES)] = vy_
                siv2[pl.ds(base, LANES)] = 1.0 / v2
                slen[pl.ds(base, LANES)] = ln
                scum[pl.ds(base, LANES)] = (off + cs) - ln
                sil[pl.ds(base, LANES)] = 1.0 / jnp.maximum(ln, EPS)
                off = off + jnp.sum(ln)
            return carry

        lax.fori_loop(0, PAIRS, prep_pair, 0)

        cp_pts.wait()
        cp_gap.wait()
        cp_ttc.wait()
        cp_feas.wait()
        cp_const.wait()

        def block_body(blk, carry):
            i0 = 2 * blk
            k = wid * PER_W + i0
            pair = k // CHUNKS
            chunk = k - pair * CHUNKS
            b = pair // A
            a = pair - b * A
            sbase = pair * SEGP
            pbase = b * NT + chunk * LANES

            def pload(ch, ofs):
                return pts_v[pl.ds(ch * POFF + pbase + ofs, LANES)]

            pxa = pload(0, 0)
            pya = pload(1, 0)
            qxa = pxa + pload(2, 0)
            qya = pya + pload(3, 0)
            vxa = pload(4, 0)
            vya = pload(5, 0)
            pxb = pload(0, LANES)
            pyb = pload(1, LANES)
            qxb = pxb + pload(2, LANES)
            qyb = pyb + pload(3, LANES)
            vxb = pload(4, LANES)
            vyb = pload(5, LANES)
            spda = _nsqrt(vxa * vxa + vya * vya + 1e-12)
            spdb = _nsqrt(vxb * vxb + vyb * vyb + 1e-12)

            sb_v = jnp.full((LANES,), sbase, jnp.int32)
            kinit = jnp.full((LANES,), KEY_BIG, jnp.int32)
            zi = jnp.zeros((LANES,), jnp.int32)

            def seg_body(j, carry_s):
                b0a, b1a, b0b, b1b, qv, jv = carry_s
                ax = plsc.load_gather(sp0x, [qv])
                ay = plsc.load_gather(sp0y, [qv])
                ux = plsc.load_gather(svx, [qv])
                uy = plsc.load_gather(svy, [qv])
                iv = plsc.load_gather(siv2, [qv])

                def upd(px, py, best):
                    wx = px - ax
                    wy = py - ay
                    t = jnp.clip((wx * ux + wy * uy) * iv, 0.0, 1.0)
                    ex = wx - t * ux
                    ey = wy - t * uy
                    d2 = ex * ex + ey * ey
                    kk = (lax.bitcast_convert_type(d2, jnp.int32) & KEY_MASK) | jv
                    return jnp.minimum(best, kk)

                return (upd(pxa, pya, b0a), upd(qxa, qya, b1a),
                        upd(pxb, pyb, b0b), upd(qxb, qyb, b1b),
                        qv + 1, jv + 1)

            b0a, b1a, b0b, b1b, _, _ = lax.fori_loop(
                0, NSEG, seg_body, (kinit, kinit, kinit, kinit, sb_v, zi),
                unroll=7)

            af = jnp.full((LANES,), a, jnp.int32)
            cvf = plsc.load_gather(consts_v, [af + 40])
            a4 = af * 4
            w0 = plsc.load_gather(consts_v, [a4])
            w1 = plsc.load_gather(consts_v, [a4 + 1])
            w2 = plsc.load_gather(consts_v, [a4 + 2])
            w3 = plsc.load_gather(consts_v, [a4 + 3])
            ld = plsc.load_gather(consts_v, [af + 32])
            zero = jnp.zeros((LANES,), jnp.float32)
            neg15 = jnp.full((LANES,), -1.5, jnp.float32)

            def finish(ii, px, py, qx, qy, vxp, vyp, speed, b0, b1):
                bj0 = b0 & 63
                bj1 = b1 & 63
                d1sq = jnp.maximum(
                    lax.bitcast_convert_type(b1 & KEY_MASK, jnp.float32), EPS)

                g0 = sb_v + bj0
                ax0 = plsc.load_gather(sp0x, [g0])
                ay0 = plsc.load_gather(sp0y, [g0])
                ux0 = plsc.load_gather(svx, [g0])
                uy0 = plsc.load_gather(svy, [g0])
                iv0 = plsc.load_gather(siv2, [g0])
                ln0 = plsc.load_gather(slen, [g0])
                cm0 = plsc.load_gather(scum, [g0])
                t0 = jnp.clip(((px - ax0) * ux0 + (py - ay0) * uy0) * iv0,
                              0.0, 1.0)
                s0 = cm0 + t0 * ln0

                g1 = sb_v + bj1
                ax1 = plsc.load_gather(sp0x, [g1])
                ay1 = plsc.load_gather(sp0y, [g1])
                ux1 = plsc.load_gather(svx, [g1])
                uy1 = plsc.load_gather(svy, [g1])
                iv1 = plsc.load_gather(siv2, [g1])
                ln1 = plsc.load_gather(slen, [g1])
                cm1 = plsc.load_gather(scum, [g1])
                il1 = plsc.load_gather(sil, [g1])
                t1 = jnp.clip(((qx - ax1) * ux1 + (qy - ay1) * uy1) * iv1,
                              0.0, 1.0)
                s1 = cm1 + t1 * ln1

                tanx = ux1 * il1
                tany = uy1 * il1
                v_along = vxp * tanx + vyp * tany
                e_s = (s1 - s0) - speed * DT
                e_v = v_along - speed

                lg = gap_v[pl.ds(ii * LANES, LANES)] * 50.0
                lt = ttc_v[pl.ds(ii * LANES, LANES)] * 5.0
                a_stop = jnp.where(speed > 0.5, neg15, zero)
                a_follow = jnp.clip(0.3 * (lg - (1.5 * speed + 2.0)), -4.0, 2.0)
                a_yield = jnp.where(lt < 2.0, neg15, zero)
                ab = jnp.where(cvf == 1.0, a_stop, zero)
                ab = jnp.where(cvf == 2.0, a_follow, ab)
                ab = jnp.where(cvf == 3.0, a_yield, ab)
                ab = jnp.clip(ab, -4.0, 2.0)

                quad = (e_s * e_s * w0 + d1sq * w1 + e_v * e_v * w2
                        + ab * ab * w3)
                lp = -0.5 * (quad + ld + C4)
                fv = feas_v[pl.ds(ii * LANES, LANES)]
                out_v[pl.ds(ii * LANES, LANES)] = jnp.where(
                    fv > 0.5, lp, jnp.full((LANES,), -1e4, jnp.float32))

            finish(i0, pxa, pya, qxa, qya, vxa, vya, spda, b0a, b1a)
            finish(i0 + 1, pxb, pyb, qxb, qyb, vxb, vyb, spdb, b0b, b1b)
            return carry

        lax.fori_loop(0, PER_W // 2, block_body, 0)
        pltpu.sync_copy(out_v, out_hbm.at[pl.ds(wbase, PTS_W)])

    return sc_kernel


def kernel(x, ctx, feasible_actions, action_path_type, action_constraint_type,
           comparable_metrics, path_polyline_idx, map_polylines, w_by_family,
           sigma):
    B, N, T, _ = x.shape
    A = action_path_type.shape[0]
    _, M, L, _ = map_polylines.shape
    NT = N * T
    LP = 128

    pts = jnp.concatenate([
        ctx[..., 0].reshape(-1),
        ctx[..., 1].reshape(-1),
        x[..., 0].reshape(-1),
        x[..., 1].reshape(-1),
        ctx[..., 3].reshape(-1),
        ctx[..., 4].reshape(-1),
    ])
    aux = jnp.concatenate([
        comparable_metrics[..., 1].transpose(0, 3, 1, 2).reshape(-1),
        comparable_metrics[..., 2].transpose(0, 3, 1, 2).reshape(-1),
        feasible_actions.transpose(0, 3, 1, 2).reshape(-1).astype(jnp.float32),
    ])

    mx = map_polylines[..., 0].reshape(B * M, L)
    my = map_polylines[..., 1].reshape(B * M, L)
    mx = jnp.concatenate([mx, jnp.repeat(mx[:, -1:], LP - L, axis=1)], axis=1)
    my = jnp.concatenate([my, jnp.repeat(my[:, -1:], LP - L, axis=1)], axis=1)

    pidx = jnp.zeros((32,), jnp.int32).at[: B * A].set(
        path_polyline_idx.reshape(-1).astype(jnp.int32))

    w = w_by_family[action_path_type]
    var = (sigma ** 2)[None, :] / jnp.maximum(w, 1e-6)
    inv_var = 1.0 / jnp.maximum(var, 1e-12)
    log_det = jnp.log(jnp.maximum(var, 1e-12)).sum(-1)
    consts = (jnp.zeros((64,), jnp.float32)
              .at[: A * 4].set(inv_var.reshape(-1))
              .at[32 : 32 + A].set(log_det)
              .at[40 : 40 + A].set(action_constraint_type.astype(jnp.float32)))

    sc = _make_sc_kernel(B, A, NT, M, L, LP)
    out = sc(mx, my, pidx, pts, aux, consts)
    return out.reshape(B, A, N, T).transpose(0, 2, 3, 1)


# speedup vs baseline: 1.5244x; 1.5244x over previous
import functools
import math

import jax
import jax.numpy as jnp
from jax import lax
from jax.experimental import pallas as pl
from jax.experimental.pallas import tpu as pltpu
from jax.experimental.pallas import tpu_sc as plsc

DT = 0.1
EPS = 1e-8
C4 = 4.0 * math.log(2.0 * math.pi)
NW = 32
LANES = 16
KEY_MASK = -64
KEY_BIG = 0x7E000000


def _nsqrt(x):
    i = lax.bitcast_convert_type(x, jnp.int32)
    i = jnp.int32(0x5F3759DF) - lax.shift_right_logical(i, 1)
    y = lax.bitcast_convert_type(i, jnp.float32)
    y = y * (1.5 - 0.5 * x * y * y)
    y = y * (1.5 - 0.5 * x * y * y)
    y = y * (1.5 - 0.5 * x * y * y)
    return x * y


def _make_sc_kernel(B, A, NT, M, L, LP):
    NSEG = L - 1
    SEGP = L
    PAIRS = B * A
    CHUNKS = NT // LANES
    ITEMS = PAIRS * CHUNKS
    PER_W = ITEMS // NW
    PTS_W = PER_W * LANES
    PIDX_PAD = 32
    POFF = B * NT
    TOT = B * A * NT
    SEGT = PAIRS * SEGP
    mesh = plsc.VectorSubcoreMesh(core_axis_name="c", subcore_axis_name="s")

    @functools.partial(
        pl.kernel,
        mesh=mesh,
        compiler_params=pltpu.CompilerParams(needs_layout_passes=False),
        out_type=jax.ShapeDtypeStruct((TOT,), jnp.float32),
        scratch_types=[
            pltpu.VMEM((PIDX_PAD,), jnp.int32),
            pltpu.VMEM((PIDX_PAD,), jnp.int32),
            pltpu.VMEM((PIDX_PAD, LP), jnp.float32),
            pltpu.VMEM((PIDX_PAD, LP), jnp.float32),
            pltpu.VMEM((SEGT,), jnp.float32),
            pltpu.VMEM((SEGT,), jnp.float32),
            pltpu.VMEM((SEGT,), jnp.float32),
            pltpu.VMEM((SEGT,), jnp.float32),
            pltpu.VMEM((SEGT,), jnp.float32),
            pltpu.VMEM((SEGT,), jnp.float32),
            pltpu.VMEM((SEGT,), jnp.float32),
            pltpu.VMEM((SEGT,), jnp.float32),
            pltpu.VMEM((6 * POFF,), jnp.float32),
            pltpu.VMEM((PTS_W,), jnp.float32),
            pltpu.VMEM((PTS_W,), jnp.float32),
            pltpu.VMEM((PTS_W,), jnp.float32),
            pltpu.VMEM((64,), jnp.float32),
            pltpu.VMEM((PTS_W,), jnp.float32),
            pltpu.SemaphoreType.DMA,
            pltpu.SemaphoreType.DMA,
        ],
    )
    def sc_kernel(map_x_hbm, map_y_hbm, pidx_hbm, pts_hbm, aux_hbm,
                  consts_hbm, out_hbm,
                  pidx_v, gidx_v, rows_x, rows_y,
                  sp0x, sp0y, svx, svy, siv2, slen, scum, sil,
                  pts_v, gap_v, ttc_v, feas_v, consts_v,
                  out_v, sem_rows, sem_bulk):
        wid = lax.axis_index("s") * 2 + lax.axis_index("c")
        wbase = wid * PTS_W

        cp_pts = pltpu.async_copy(pts_hbm, pts_v, sem_bulk)
        cp_gap = pltpu.async_copy(aux_hbm.at[pl.ds(wbase, PTS_W)], gap_v, sem_bulk)
        cp_ttc = pltpu.async_copy(aux_hbm.at[pl.ds(TOT + wbase, PTS_W)], ttc_v, sem_bulk)
        cp_feas = pltpu.async_copy(aux_hbm.at[pl.ds(2 * TOT + wbase, PTS_W)], feas_v, sem_bulk)
        cp_const = pltpu.async_copy(consts_hbm, consts_v, sem_bulk)

        pltpu.sync_copy(pidx_hbm, pidx_v)
        for c in range(PIDX_PAD // LANES):
            pr = lax.iota(jnp.int32, LANES) + (c * LANES)
            row = pidx_v[pl.ds(c * LANES, LANES)] + (pr // A) * M
            gidx_v[pl.ds(c * LANES, LANES)] = jnp.minimum(row, B * M - 1)
        cp_rx = pltpu.async_copy(map_x_hbm.at[gidx_v], rows_x, sem_rows)
        cp_ry = pltpu.async_copy(map_y_hbm.at[gidx_v], rows_y, sem_rows)
        cp_rx.wait()
        cp_ry.wait()

        def prep_pair(p, carry):
            off = jnp.float32(0.0)
            for c in range(SEGP // LANES):
                x_lo = rows_x[p, pl.ds(c * LANES, LANES)]
                x_hi = rows_x[p, pl.ds(c * LANES + 1, LANES)]
                y_lo = rows_y[p, pl.ds(c * LANES, LANES)]
                y_hi = rows_y[p, pl.ds(c * LANES + 1, LANES)]
                vx_ = x_hi - x_lo
                vy_ = y_hi - y_lo
                v2 = jnp.maximum(vx_ * vx_ + vy_ * vy_, EPS)
                ln = _nsqrt(v2)
                cs = plsc.cumsum(ln)
                base = p * SEGP + c * LANES
                sp0x[pl.ds(base, LANES)] = x_lo
                sp0y[pl.ds(base, LANES)] = y_lo
                svx[pl.ds(base, LANES)] = vx_
                svy[pl.ds(base, LANES)] = vy_
                siv2[pl.ds(base, LANES)] = 1.0 / v2
                slen[pl.ds(base, LANES)] = ln
                scum[pl.ds(base, LANES)] = (off + cs) - ln
                sil[pl.ds(base, LANES)] = 1.0 / jnp.maximum(ln, EPS)
                off = off + jnp.sum(ln)
            return carry

        lax.fori_loop(0, PAIRS, prep_pair, 0)

        cp_pts.wait()
        cp_gap.wait()
        cp_ttc.wait()
        cp_feas.wait()
        cp_const.wait()

        def block_body(blk, carry):
            i0 = 2 * blk
            k = wid * PER_W + i0
            pair = k // CHUNKS
            chunk = k - pair * CHUNKS
            b = pair // A
            a = pair - b * A
            sbase = pair * SEGP
            pbase = b * NT + chunk * LANES

            def pload(ch, ofs):
                return pts_v[pl.ds(ch * POFF + pbase + ofs, LANES)]

            pxa = pload(0, 0)
            pya = pload(1, 0)
            qxa = pxa + pload(2, 0)
            qya = pya + pload(3, 0)
            vxa = pload(4, 0)
            vya = pload(5, 0)
            pxb = pload(0, LANES)
            pyb = pload(1, LANES)
            qxb = pxb + pload(2, LANES)
            qyb = pyb + pload(3, LANES)
            vxb = pload(4, LANES)
            vyb = pload(5, LANES)
            spda = _nsqrt(vxa * vxa + vya * vya + 1e-12)
            spdb = _nsqrt(vxb * vxb + vyb * vyb + 1e-12)

            sb_v = jnp.full((LANES,), sbase, jnp.int32)
            kinit = jnp.full((LANES,), KEY_BIG, jnp.int32)
            zi = jnp.zeros((LANES,), jnp.int32)

            def seg_body(j, carry_s):
                b0a, b1a, b0b, b1b, qv, jv = carry_s
                ax = plsc.load_gather(sp0x, [qv])
                ay = plsc.load_gather(sp0y, [qv])
                ux = plsc.load_gather(svx, [qv])
                uy = plsc.load_gather(svy, [qv])
                iv = plsc.load_gather(siv2, [qv])

                def upd(px, py, best):
                    wx = px - ax
                    wy = py - ay
                    t = jnp.clip((wx * ux + wy * uy) * iv, 0.0, 1.0)
                    ex = wx - t * ux
                    ey = wy - t * uy
                    d2 = ex * ex + ey * ey
                    kk = (lax.bitcast_convert_type(d2, jnp.int32) & KEY_MASK) | jv
                    return jnp.minimum(best, kk)

                return (upd(pxa, pya, b0a), upd(qxa, qya, b1a),
                        upd(pxb, pyb, b0b), upd(qxb, qyb, b1b),
                        qv + 1, jv + 1)

            b0a, b1a, b0b, b1b, _, _ = lax.fori_loop(
                0, NSEG, seg_body, (kinit, kinit, kinit, kinit, sb_v, zi),
                unroll=3)

            af = jnp.full((LANES,), a, jnp.int32)
            cvf = plsc.load_gather(consts_v, [af + 40])
            a4 = af * 4
            w0 = plsc.load_gather(consts_v, [a4])
            w1 = plsc.load_gather(consts_v, [a4 + 1])
            w2 = plsc.load_gather(consts_v, [a4 + 2])
            w3 = plsc.load_gather(consts_v, [a4 + 3])
            ld = plsc.load_gather(consts_v, [af + 32])
            zero = jnp.zeros((LANES,), jnp.float32)
            neg15 = jnp.full((LANES,), -1.5, jnp.float32)

            def finish(ii, px, py, qx, qy, vxp, vyp, speed, b0, b1):
                bj0 = b0 & 63
                bj1 = b1 & 63
                d1sq = jnp.maximum(
                    lax.bitcast_convert_type(b1 & KEY_MASK, jnp.float32), EPS)

                g0 = sb_v + bj0
                ax0 = plsc.load_gather(sp0x, [g0])
                ay0 = plsc.load_gather(sp0y, [g0])
                ux0 = plsc.load_gather(svx, [g0])
                uy0 = plsc.load_gather(svy, [g0])
                iv0 = plsc.load_gather(siv2, [g0])
                ln0 = plsc.load_gather(slen, [g0])
                cm0 = plsc.load_gather(scum, [g0])
                t0 = jnp.clip(((px - ax0) * ux0 + (py - ay0) * uy0) * iv0,
                              0.0, 1.0)
                s0 = cm0 + t0 * ln0

                g1 = sb_v + bj1
                ax1 = plsc.load_gather(sp0x, [g1])
                ay1 = plsc.load_gather(sp0y, [g1])
                ux1 = plsc.load_gather(svx, [g1])
                uy1 = plsc.load_gather(svy, [g1])
                iv1 = plsc.load_gather(siv2, [g1])
                ln1 = plsc.load_gather(slen, [g1])
                cm1 = plsc.load_gather(scum, [g1])
                il1 = plsc.load_gather(sil, [g1])
                t1 = jnp.clip(((qx - ax1) * ux1 + (qy - ay1) * uy1) * iv1,
                              0.0, 1.0)
                s1 = cm1 + t1 * ln1

                tanx = ux1 * il1
                tany = uy1 * il1
                v_along = vxp * tanx + vyp * tany
                e_s = (s1 - s0) - speed * DT
                e_v = v_along - speed

                lg = gap_v[pl.ds(ii * LANES, LANES)] * 50.0
                lt = ttc_v[pl.ds(ii * LANES, LANES)] * 5.0
                a_stop = jnp.where(speed > 0.5, neg15, zero)
                a_follow = jnp.clip(0.3 * (lg - (1.5 * speed + 2.0)), -4.0, 2.0)
                a_yield = jnp.where(lt < 2.0, neg15, zero)
                ab = jnp.where(cvf == 1.0, a_stop, zero)
                ab = jnp.where(cvf == 2.0, a_follow, ab)
                ab = jnp.where(cvf == 3.0, a_yield, ab)
                ab = jnp.clip(ab, -4.0, 2.0)

                quad = (e_s * e_s * w0 + d1sq * w1 + e_v * e_v * w2
                        + ab * ab * w3)
                lp = -0.5 * (quad + ld + C4)
                fv = feas_v[pl.ds(ii * LANES, LANES)]
                out_v[pl.ds(ii * LANES, LANES)] = jnp.where(
                    fv > 0.5, lp, jnp.full((LANES,), -1e4, jnp.float32))

            finish(i0, pxa, pya, qxa, qya, vxa, vya, spda, b0a, b1a)
            finish(i0 + 1, pxb, pyb, qxb, qyb, vxb, vyb, spdb, b0b, b1b)
            return carry

        lax.fori_loop(0, PER_W // 2, block_body, 0)
        pltpu.sync_copy(out_v, out_hbm.at[pl.ds(wbase, PTS_W)])

    return sc_kernel


def kernel(x, ctx, feasible_actions, action_path_type, action_constraint_type,
           comparable_metrics, path_polyline_idx, map_polylines, w_by_family,
           sigma):
    B, N, T, _ = x.shape
    A = action_path_type.shape[0]
    _, M, L, _ = map_polylines.shape
    NT = N * T
    LP = 128

    pts = jnp.concatenate([
        ctx[..., 0].reshape(-1),
        ctx[..., 1].reshape(-1),
        x[..., 0].reshape(-1),
        x[..., 1].reshape(-1),
        ctx[..., 3].reshape(-1),
        ctx[..., 4].reshape(-1),
    ])
    aux = jnp.concatenate([
        comparable_metrics[..., 1].transpose(0, 3, 1, 2).reshape(-1),
        comparable_metrics[..., 2].transpose(0, 3, 1, 2).reshape(-1),
        feasible_actions.transpose(0, 3, 1, 2).reshape(-1).astype(jnp.float32),
    ])

    mx = map_polylines[..., 0].reshape(B * M, L)
    my = map_polylines[..., 1].reshape(B * M, L)
    mx = jnp.concatenate([mx, jnp.repeat(mx[:, -1:], LP - L, axis=1)], axis=1)
    my = jnp.concatenate([my, jnp.repeat(my[:, -1:], LP - L, axis=1)], axis=1)

    pidx = jnp.zeros((32,), jnp.int32).at[: B * A].set(
        path_polyline_idx.reshape(-1).astype(jnp.int32))

    w = w_by_family[action_path_type]
    var = (sigma ** 2)[None, :] / jnp.maximum(w, 1e-6)
    inv_var = 1.0 / jnp.maximum(var, 1e-12)
    log_det = jnp.log(jnp.maximum(var, 1e-12)).sum(-1)
    consts = (jnp.zeros((64,), jnp.float32)
              .at[: A * 4].set(inv_var.reshape(-1))
              .at[32 : 32 + A].set(log_det)
              .at[40 : 40 + A].set(action_constraint_type.astype(jnp.float32)))

    sc = _make_sc_kernel(B, A, NT, M, L, LP)
    out = sc(mx, my, pidx, pts, aux, consts)
    return out.reshape(B, A, N, T).transpose(0, 2, 3, 1)


# speedup vs baseline: 1.6102x; 1.0563x over previous
import functools
import math

import jax
import jax.numpy as jnp
from jax import lax
from jax.experimental import pallas as pl
from jax.experimental.pallas import tpu as pltpu
from jax.experimental.pallas import tpu_sc as plsc

DT = 0.1
EPS = 1e-8
C4 = 4.0 * math.log(2.0 * math.pi)
NW = 32
LANES = 16
KEY_MASK = -64
KEY_BIG = 0x7E000000


def _nsqrt(x):
    i = lax.bitcast_convert_type(x, jnp.int32)
    i = jnp.int32(0x5F3759DF) - lax.shift_right_logical(i, 1)
    y = lax.bitcast_convert_type(i, jnp.float32)
    y = y * (1.5 - 0.5 * x * y * y)
    y = y * (1.5 - 0.5 * x * y * y)
    y = y * (1.5 - 0.5 * x * y * y)
    return x * y


def _make_sc_kernel(B, A, NT, M, L, LP):
    NSEG = L - 1
    SEGP = L
    PAIRS = B * A
    CHUNKS = NT // LANES
    ITEMS = PAIRS * CHUNKS
    PER_W = ITEMS // NW
    PTS_W = PER_W * LANES
    PIDX_PAD = 32
    POFF = B * NT
    TOT = B * A * NT
    SEGT = PAIRS * SEGP
    mesh = plsc.VectorSubcoreMesh(core_axis_name="c", subcore_axis_name="s")

    @functools.partial(
        pl.kernel,
        mesh=mesh,
        compiler_params=pltpu.CompilerParams(needs_layout_passes=False),
        out_type=jax.ShapeDtypeStruct((TOT,), jnp.float32),
        scratch_types=[
            pltpu.VMEM((PIDX_PAD,), jnp.int32),
            pltpu.VMEM((PIDX_PAD,), jnp.int32),
            pltpu.VMEM((PIDX_PAD, LP), jnp.float32),
            pltpu.VMEM((PIDX_PAD, LP), jnp.float32),
            pltpu.VMEM((2 * SEGP,), jnp.float32),
            pltpu.VMEM((2 * SEGP,), jnp.float32),
            pltpu.VMEM((2 * SEGP,), jnp.float32),
            pltpu.VMEM((2 * SEGP,), jnp.float32),
            pltpu.VMEM((2 * SEGP,), jnp.float32),
            pltpu.VMEM((2 * SEGP,), jnp.float32),
            pltpu.VMEM((2 * SEGP,), jnp.float32),
            pltpu.VMEM((2 * SEGP,), jnp.float32),
            pltpu.VMEM((2 * 6 * NT,), jnp.float32),
            pltpu.VMEM((PTS_W,), jnp.float32),
            pltpu.VMEM((PTS_W,), jnp.float32),
            pltpu.VMEM((PTS_W,), jnp.float32),
            pltpu.VMEM((64,), jnp.float32),
            pltpu.VMEM((PTS_W,), jnp.float32),
            pltpu.SemaphoreType.DMA,
            pltpu.SemaphoreType.DMA,
        ],
    )
    def sc_kernel(map_x_hbm, map_y_hbm, pidx_hbm, pts_hbm, aux_hbm,
                  consts_hbm, out_hbm,
                  pidx_v, gidx_v, rows_x, rows_y,
                  sp0x, sp0y, svx, svy, siv2, slen, scum, sil,
                  pts_v, gap_v, ttc_v, feas_v, consts_v,
                  out_v, sem_rows, sem_bulk):
        wid = lax.axis_index("s") * 2 + lax.axis_index("c")
        wbase = wid * PTS_W
        k0 = wid * PER_W
        p_lo = k0 // CHUNKS
        p_hi = (k0 + PER_W - 1) // CHUNKS

        cp_bulk = []
        for s in range(2):
            ps = jnp.minimum(p_lo + s, p_hi)
            bs = ps // A
            for ch in range(6):
                cp_bulk.append(pltpu.async_copy(
                    pts_hbm.at[pl.ds(ch * POFF + bs * NT, NT)],
                    pts_v.at[pl.ds((s * 6 + ch) * NT, NT)], sem_bulk))
        cp_bulk.append(pltpu.async_copy(
            aux_hbm.at[pl.ds(wbase, PTS_W)], gap_v, sem_bulk))
        cp_bulk.append(pltpu.async_copy(
            aux_hbm.at[pl.ds(TOT + wbase, PTS_W)], ttc_v, sem_bulk))
        cp_bulk.append(pltpu.async_copy(
            aux_hbm.at[pl.ds(2 * TOT + wbase, PTS_W)], feas_v, sem_bulk))
        cp_bulk.append(pltpu.async_copy(consts_hbm, consts_v, sem_bulk))

        pltpu.sync_copy(pidx_hbm, pidx_v)
        for c in range(PIDX_PAD // LANES):
            pr = lax.iota(jnp.int32, LANES) + (c * LANES)
            row = pidx_v[pl.ds(c * LANES, LANES)] + (pr // A) * M
            gidx_v[pl.ds(c * LANES, LANES)] = jnp.minimum(row, B * M - 1)
        cp_rx = pltpu.async_copy(map_x_hbm.at[gidx_v], rows_x, sem_rows)
        cp_ry = pltpu.async_copy(map_y_hbm.at[gidx_v], rows_y, sem_rows)
        cp_rx.wait()
        cp_ry.wait()

        def prep_slot(s, carry):
            p = jnp.minimum(p_lo + s, p_hi)
            off = jnp.float32(0.0)
            for c in range(SEGP // LANES):
                x_lo = rows_x[p, pl.ds(c * LANES, LANES)]
                x_hi = rows_x[p, pl.ds(c * LANES + 1, LANES)]
                y_lo = rows_y[p, pl.ds(c * LANES, LANES)]
                y_hi = rows_y[p, pl.ds(c * LANES + 1, LANES)]
                vx_ = x_hi - x_lo
                vy_ = y_hi - y_lo
                v2 = jnp.maximum(vx_ * vx_ + vy_ * vy_, EPS)
                ln = _nsqrt(v2)
                cs = plsc.cumsum(ln)
                base = s * SEGP + c * LANES
                sp0x[pl.ds(base, LANES)] = x_lo
                sp0y[pl.ds(base, LANES)] = y_lo
                svx[pl.ds(base, LANES)] = vx_
                svy[pl.ds(base, LANES)] = vy_
                siv2[pl.ds(base, LANES)] = 1.0 / v2
                slen[pl.ds(base, LANES)] = ln
                scum[pl.ds(base, LANES)] = (off + cs) - ln
                sil[pl.ds(base, LANES)] = 1.0 / jnp.maximum(ln, EPS)
                off = off + jnp.sum(ln)
            return carry

        lax.fori_loop(0, 2, prep_slot, 0)

        for cp in cp_bulk:
            cp.wait()

        def block_body(blk, carry):
            i0 = 2 * blk
            k = k0 + i0
            pair = k // CHUNKS
            chunk = k - pair * CHUNKS
            a = pair - (pair // A) * A
            slot = pair - p_lo
            sbase = slot * SEGP
            pbase = slot * (6 * NT) + chunk * LANES

            def pload(ch, ofs):
                return pts_v[pl.ds(ch * NT + pbase + ofs, LANES)]

            pxa = pload(0, 0)
            pya = pload(1, 0)
            qxa = pxa + pload(2, 0)
            qya = pya + pload(3, 0)
            vxa = pload(4, 0)
            vya = pload(5, 0)
            pxb = pload(0, LANES)
            pyb = pload(1, LANES)
            qxb = pxb + pload(2, LANES)
            qyb = pyb + pload(3, LANES)
            vxb = pload(4, LANES)
            vyb = pload(5, LANES)
            spda = _nsqrt(vxa * vxa + vya * vya + 1e-12)
            spdb = _nsqrt(vxb * vxb + vyb * vyb + 1e-12)

            sb_v = jnp.full((LANES,), sbase, jnp.int32)
            kinit = jnp.full((LANES,), KEY_BIG, jnp.int32)
            zi = jnp.zeros((LANES,), jnp.int32)

            def seg_body(j, carry_s):
                b0a, b1a, b0b, b1b, qv, jv = carry_s
                ax = plsc.load_gather(sp0x, [qv])
                ay = plsc.load_gather(sp0y, [qv])
                ux = plsc.load_gather(svx, [qv])
                uy = plsc.load_gather(svy, [qv])
                iv = plsc.load_gather(siv2, [qv])

                def upd(px, py, best):
                    wx = px - ax
                    wy = py - ay
                    t = jnp.clip((wx * ux + wy * uy) * iv, 0.0, 1.0)
                    ex = wx - t * ux
                    ey = wy - t * uy
                    d2 = ex * ex + ey * ey
                    kk = (lax.bitcast_convert_type(d2, jnp.int32) & KEY_MASK) | jv
                    return jnp.minimum(best, kk)

                return (upd(pxa, pya, b0a), upd(qxa, qya, b1a),
                        upd(pxb, pyb, b0b), upd(qxb, qyb, b1b),
                        qv + 1, jv + 1)

            b0a, b1a, b0b, b1b, _, _ = lax.fori_loop(
                0, NSEG, seg_body, (kinit, kinit, kinit, kinit, sb_v, zi),
                unroll=3)

            af = jnp.full((LANES,), a, jnp.int32)
            cvf = plsc.load_gather(consts_v, [af + 40])
            a4 = af * 4
            w0 = plsc.load_gather(consts_v, [a4])
            w1 = plsc.load_gather(consts_v, [a4 + 1])
            w2 = plsc.load_gather(consts_v, [a4 + 2])
            w3 = plsc.load_gather(consts_v, [a4 + 3])
            ld = plsc.load_gather(consts_v, [af + 32])
            zero = jnp.zeros((LANES,), jnp.float32)
            neg15 = jnp.full((LANES,), -1.5, jnp.float32)

            def finish(ii, px, py, qx, qy, vxp, vyp, speed, b0, b1):
                bj0 = b0 & 63
                bj1 = b1 & 63
                d1sq = jnp.maximum(
                    lax.bitcast_convert_type(b1 & KEY_MASK, jnp.float32), EPS)

                g0 = sb_v + bj0
                ax0 = plsc.load_gather(sp0x, [g0])
                ay0 = plsc.load_gather(sp0y, [g0])
                ux0 = plsc.load_gather(svx, [g0])
                uy0 = plsc.load_gather(svy, [g0])
                iv0 = plsc.load_gather(siv2, [g0])
                ln0 = plsc.load_gather(slen, [g0])
                cm0 = plsc.load_gather(scum, [g0])
                t0 = jnp.clip(((px - ax0) * ux0 + (py - ay0) * uy0) * iv0,
                              0.0, 1.0)
                s0 = cm0 + t0 * ln0

                g1 = sb_v + bj1
                ax1 = plsc.load_gather(sp0x, [g1])
                ay1 = plsc.load_gather(sp0y, [g1])
                ux1 = plsc.load_gather(svx, [g1])
                uy1 = plsc.load_gather(svy, [g1])
                iv1 = plsc.load_gather(siv2, [g1])
                ln1 = plsc.load_gather(slen, [g1])
                cm1 = plsc.load_gather(scum, [g1])
                il1 = plsc.load_gather(sil, [g1])
                t1 = jnp.clip(((qx - ax1) * ux1 + (qy - ay1) * uy1) * iv1,
                              0.0, 1.0)
                s1 = cm1 + t1 * ln1

                tanx = ux1 * il1
                tany = uy1 * il1
                v_along = vxp * tanx + vyp * tany
                e_s = (s1 - s0) - speed * DT
                e_v = v_along - speed

                lg = gap_v[pl.ds(ii * LANES, LANES)] * 50.0
                lt = ttc_v[pl.ds(ii * LANES, LANES)] * 5.0
                a_stop = jnp.where(speed > 0.5, neg15, zero)
                a_follow = jnp.clip(0.3 * (lg - (1.5 * speed + 2.0)), -4.0, 2.0)
                a_yield = jnp.where(lt < 2.0, neg15, zero)
                ab = jnp.where(cvf == 1.0, a_stop, zero)
                ab = jnp.where(cvf == 2.0, a_follow, ab)
                ab = jnp.where(cvf == 3.0, a_yield, ab)
                ab = jnp.clip(ab, -4.0, 2.0)

                quad = (e_s * e_s * w0 + d1sq * w1 + e_v * e_v * w2
                        + ab * ab * w3)
                lp = -0.5 * (quad + ld + C4)
                fv = feas_v[pl.ds(ii * LANES, LANES)]
                out_v[pl.ds(ii * LANES, LANES)] = jnp.where(
                    fv > 0.5, lp, jnp.full((LANES,), -1e4, jnp.float32))

            finish(i0, pxa, pya, qxa, qya, vxa, vya, spda, b0a, b1a)
            finish(i0 + 1, pxb, pyb, qxb, qyb, vxb, vyb, spdb, b0b, b1b)
            return carry

        lax.fori_loop(0, PER_W // 2, block_body, 0)
        pltpu.sync_copy(out_v, out_hbm.at[pl.ds(wbase, PTS_W)])

    return sc_kernel


def kernel(x, ctx, feasible_actions, action_path_type, action_constraint_type,
           comparable_metrics, path_polyline_idx, map_polylines, w_by_family,
           sigma):
    B, N, T, _ = x.shape
    A = action_path_type.shape[0]
    _, M, L, _ = map_polylines.shape
    NT = N * T
    LP = 128

    pts = jnp.concatenate([
        ctx[..., 0].reshape(-1),
        ctx[..., 1].reshape(-1),
        x[..., 0].reshape(-1),
        x[..., 1].reshape(-1),
        ctx[..., 3].reshape(-1),
        ctx[..., 4].reshape(-1),
    ])
    aux = jnp.concatenate([
        comparable_metrics[..., 1].transpose(0, 3, 1, 2).reshape(-1),
        comparable_metrics[..., 2].transpose(0, 3, 1, 2).reshape(-1),
        feasible_actions.transpose(0, 3, 1, 2).reshape(-1).astype(jnp.float32),
    ])

    mx = map_polylines[..., 0].reshape(B * M, L)
    my = map_polylines[..., 1].reshape(B * M, L)
    mx = jnp.concatenate([mx, jnp.repeat(mx[:, -1:], LP - L, axis=1)], axis=1)
    my = jnp.concatenate([my, jnp.repeat(my[:, -1:], LP - L, axis=1)], axis=1)

    pidx = jnp.zeros((32,), jnp.int32).at[: B * A].set(
        path_polyline_idx.reshape(-1).astype(jnp.int32))

    w = w_by_family[action_path_type]
    var = (sigma ** 2)[None, :] / jnp.maximum(w, 1e-6)
    inv_var = 1.0 / jnp.maximum(var, 1e-12)
    log_det = jnp.log(jnp.maximum(var, 1e-12)).sum(-1)
    consts = (jnp.zeros((64,), jnp.float32)
              .at[: A * 4].set(inv_var.reshape(-1))
              .at[32 : 32 + A].set(log_det)
              .at[40 : 40 + A].set(action_constraint_type.astype(jnp.float32)))

    sc = _make_sc_kernel(B, A, NT, M, L, LP)
    out = sc(mx, my, pidx, pts, aux, consts)
    return out.reshape(B, A, N, T).transpose(0, 2, 3, 1)


# speedup vs baseline: 1.6312x; 1.0130x over previous
import functools
import math

import jax
import jax.numpy as jnp
from jax import lax
from jax.experimental import pallas as pl
from jax.experimental.pallas import tpu as pltpu
from jax.experimental.pallas import tpu_sc as plsc

DT = 0.1
EPS = 1e-8
C4 = 4.0 * math.log(2.0 * math.pi)
NW = 32
LANES = 16
KEY_MASK = -64
KEY_BIG = 0x7E000000


def _nsqrt(x):
    i = lax.bitcast_convert_type(x, jnp.int32)
    i = jnp.int32(0x5F3759DF) - lax.shift_right_logical(i, 1)
    y = lax.bitcast_convert_type(i, jnp.float32)
    y = y * (1.5 - 0.5 * x * y * y)
    y = y * (1.5 - 0.5 * x * y * y)
    y = y * (1.5 - 0.5 * x * y * y)
    return x * y


def _make_sc_kernel(B, A, NT, M, L, LP):
    NSEG = L - 1
    SEGP = L
    PAIRS = B * A
    CHUNKS = NT // LANES
    ITEMS = PAIRS * CHUNKS
    PER_W = ITEMS // NW
    PTS_W = PER_W * LANES
    PIDX_PAD = 32
    POFF = B * NT
    TOT = B * A * NT
    SEGT = PAIRS * SEGP
    mesh = plsc.VectorSubcoreMesh(core_axis_name="c", subcore_axis_name="s")

    @functools.partial(
        pl.kernel,
        mesh=mesh,
        compiler_params=pltpu.CompilerParams(needs_layout_passes=False),
        out_type=jax.ShapeDtypeStruct((TOT,), jnp.float32),
        scratch_types=[
            pltpu.VMEM((PIDX_PAD,), jnp.int32),
            pltpu.VMEM((PIDX_PAD,), jnp.int32),
            pltpu.VMEM((PIDX_PAD, LP), jnp.float32),
            pltpu.VMEM((PIDX_PAD, LP), jnp.float32),
            pltpu.VMEM((2 * SEGP,), jnp.float32),
            pltpu.VMEM((2 * SEGP,), jnp.float32),
            pltpu.VMEM((2 * SEGP,), jnp.float32),
            pltpu.VMEM((2 * SEGP,), jnp.float32),
            pltpu.VMEM((2 * SEGP,), jnp.float32),
            pltpu.VMEM((2 * SEGP,), jnp.float32),
            pltpu.VMEM((2 * SEGP,), jnp.float32),
            pltpu.VMEM((2 * SEGP,), jnp.float32),
            pltpu.VMEM((2 * 6 * NT,), jnp.float32),
            pltpu.VMEM((PTS_W,), jnp.float32),
            pltpu.VMEM((PTS_W,), jnp.float32),
            pltpu.VMEM((PTS_W,), jnp.float32),
            pltpu.VMEM((64,), jnp.float32),
            pltpu.VMEM((PTS_W,), jnp.float32),
            pltpu.SemaphoreType.DMA,
            pltpu.SemaphoreType.DMA,
        ],
    )
    def sc_kernel(map_x_hbm, map_y_hbm, pidx_hbm, pts_hbm, aux_hbm,
                  consts_hbm, out_hbm,
                  pidx_v, gidx_v, rows_x, rows_y,
                  sp0x, sp0y, svx, svy, siv2, slen, scum, sil,
                  pts_v, gap_v, ttc_v, feas_v, consts_v,
                  out_v, sem_rows, sem_bulk):
        wid = lax.axis_index("s") * 2 + lax.axis_index("c")
        wbase = wid * PTS_W
        k0 = wid * PER_W
        p_lo = k0 // CHUNKS
        p_hi = (k0 + PER_W - 1) // CHUNKS

        cp_bulk = []
        for s in range(2):
            ps = jnp.minimum(p_lo + s, p_hi)
            bs = ps // A
            for ch in range(6):
                cp_bulk.append(pltpu.async_copy(
                    pts_hbm.at[pl.ds(ch * POFF + bs * NT, NT)],
                    pts_v.at[pl.ds((s * 6 + ch) * NT, NT)], sem_bulk))
        cp_bulk.append(pltpu.async_copy(
            aux_hbm.at[pl.ds(wbase, PTS_W)], gap_v, sem_bulk))
        cp_bulk.append(pltpu.async_copy(
            aux_hbm.at[pl.ds(TOT + wbase, PTS_W)], ttc_v, sem_bulk))
        cp_bulk.append(pltpu.async_copy(
            aux_hbm.at[pl.ds(2 * TOT + wbase, PTS_W)], feas_v, sem_bulk))
        cp_bulk.append(pltpu.async_copy(consts_hbm, consts_v, sem_bulk))

        pltpu.sync_copy(pidx_hbm, pidx_v)
        for c in range(PIDX_PAD // LANES):
            pr = lax.iota(jnp.int32, LANES) + (c * LANES)
            row = pidx_v[pl.ds(c * LANES, LANES)] + (pr // A) * M
            gidx_v[pl.ds(c * LANES, LANES)] = jnp.minimum(row, B * M - 1)
        cp_rx = pltpu.async_copy(map_x_hbm.at[gidx_v], rows_x, sem_rows)
        cp_ry = pltpu.async_copy(map_y_hbm.at[gidx_v], rows_y, sem_rows)
        cp_rx.wait()
        cp_ry.wait()

        def prep_slot(s, carry):
            p = jnp.minimum(p_lo + s, p_hi)
            off = jnp.float32(0.0)
            for c in range(SEGP // LANES):
                x_lo = rows_x[p, pl.ds(c * LANES, LANES)]
                x_hi = rows_x[p, pl.ds(c * LANES + 1, LANES)]
                y_lo = rows_y[p, pl.ds(c * LANES, LANES)]
                y_hi = rows_y[p, pl.ds(c * LANES + 1, LANES)]
                vx_ = x_hi - x_lo
                vy_ = y_hi - y_lo
                v2 = jnp.maximum(vx_ * vx_ + vy_ * vy_, EPS)
                ln = _nsqrt(v2)
                cs = plsc.cumsum(ln)
                iv2 = 1.0 / v2
                base = s * SEGP + c * LANES
                sp0x[pl.ds(base, LANES)] = x_lo
                sp0y[pl.ds(base, LANES)] = y_lo
                svx[pl.ds(base, LANES)] = vx_
                svy[pl.ds(base, LANES)] = vy_
                siv2[pl.ds(base, LANES)] = iv2
                slen[pl.ds(base, LANES)] = ln
                scum[pl.ds(base, LANES)] = (off + cs) - ln
                sil[pl.ds(base, LANES)] = 1.0 / jnp.maximum(ln, EPS)
                off = off + jnp.sum(ln)
            return carry

        lax.fori_loop(0, 2, prep_slot, 0)

        for cp in cp_bulk:
            cp.wait()

        def block_body(blk, carry):
            i0 = 2 * blk
            k = k0 + i0
            pair = k // CHUNKS
            chunk = k - pair * CHUNKS
            a = pair - (pair // A) * A
            slot = pair - p_lo
            sbase = slot * SEGP
            pbase = slot * (6 * NT) + chunk * LANES

            def pload(ch, ofs):
                return pts_v[pl.ds(ch * NT + pbase + ofs, LANES)]

            pxa = pload(0, 0)
            pya = pload(1, 0)
            qxa = pxa + pload(2, 0)
            qya = pya + pload(3, 0)
            vxa = pload(4, 0)
            vya = pload(5, 0)
            pxb = pload(0, LANES)
            pyb = pload(1, LANES)
            qxb = pxb + pload(2, LANES)
            qyb = pyb + pload(3, LANES)
            vxb = pload(4, LANES)
            vyb = pload(5, LANES)
            spda = _nsqrt(vxa * vxa + vya * vya + 1e-12)
            spdb = _nsqrt(vxb * vxb + vyb * vyb + 1e-12)

            sb_v = jnp.full((LANES,), sbase, jnp.int32)
            kinit = jnp.full((LANES,), KEY_BIG, jnp.int32)
            zi = jnp.zeros((LANES,), jnp.int32)

            def seg_body(j, carry_s):
                k0a, k1a, k0b, k1b, qv, jv = carry_s
                ax = plsc.load_gather(sp0x, [qv])
                ay = plsc.load_gather(sp0y, [qv])
                ux = plsc.load_gather(svx, [qv])
                uy = plsc.load_gather(svy, [qv])
                iv = plsc.load_gather(siv2, [qv])

                def upd(px, py, best):
                    wx = px - ax
                    wy = py - ay
                    t = jnp.clip((wx * ux + wy * uy) * iv, 0.0, 1.0)
                    ex = wx - t * ux
                    ey = wy - t * uy
                    d2 = ex * ex + ey * ey
                    kk = (lax.bitcast_convert_type(d2, jnp.int32) & KEY_MASK) | jv
                    return jnp.minimum(best, kk)

                return (upd(pxa, pya, k0a), upd(qxa, qya, k1a),
                        upd(pxb, pyb, k0b), upd(qxb, qyb, k1b),
                        qv + 1, jv + 1)

            k0a, k1a, k0b, k1b, _, _ = lax.fori_loop(
                0, NSEG, seg_body, (kinit, kinit, kinit, kinit, sb_v, zi),
                unroll=3)

            bj0a = k0a & 63
            bj1a = k1a & 63
            bj0b = k0b & 63
            bj1b = k1b & 63

            af = jnp.full((LANES,), a, jnp.int32)
            cvf = plsc.load_gather(consts_v, [af + 40])
            a4 = af * 4
            w0 = plsc.load_gather(consts_v, [a4])
            w1 = plsc.load_gather(consts_v, [a4 + 1])
            w2 = plsc.load_gather(consts_v, [a4 + 2])
            w3 = plsc.load_gather(consts_v, [a4 + 3])
            ld = plsc.load_gather(consts_v, [af + 32])
            zero = jnp.zeros((LANES,), jnp.float32)
            neg15 = jnp.full((LANES,), -1.5, jnp.float32)

            def finish(ii, px, py, qx, qy, vxp, vyp, speed, bj0, bj1):
                g0 = sb_v + bj0.astype(jnp.int32)
                ax0 = plsc.load_gather(sp0x, [g0])
                ay0 = plsc.load_gather(sp0y, [g0])
                ux0 = plsc.load_gather(svx, [g0])
                uy0 = plsc.load_gather(svy, [g0])
                iv0 = plsc.load_gather(siv2, [g0])
                ln0 = plsc.load_gather(slen, [g0])
                cm0 = plsc.load_gather(scum, [g0])
                t0 = jnp.clip(((px - ax0) * ux0 + (py - ay0) * uy0) * iv0,
                              0.0, 1.0)
                s0 = cm0 + t0 * ln0

                g1 = sb_v + bj1.astype(jnp.int32)
                ax1 = plsc.load_gather(sp0x, [g1])
                ay1 = plsc.load_gather(sp0y, [g1])
                ux1 = plsc.load_gather(svx, [g1])
                uy1 = plsc.load_gather(svy, [g1])
                iv1 = plsc.load_gather(siv2, [g1])
                ln1 = plsc.load_gather(slen, [g1])
                cm1 = plsc.load_gather(scum, [g1])
                il1 = plsc.load_gather(sil, [g1])
                w1x = qx - ax1
                w1y = qy - ay1
                t1 = jnp.clip((w1x * ux1 + w1y * uy1) * iv1, 0.0, 1.0)
                s1 = cm1 + t1 * ln1
                e1x = w1x - t1 * ux1
                e1y = w1y - t1 * uy1
                d1sq = jnp.maximum(e1x * e1x + e1y * e1y, EPS)

                tanx = ux1 * il1
                tany = uy1 * il1
                v_along = vxp * tanx + vyp * tany
                e_s = (s1 - s0) - speed * DT
                e_v = v_along - speed

                lg = gap_v[pl.ds(ii * LANES, LANES)] * 50.0
                lt = ttc_v[pl.ds(ii * LANES, LANES)] * 5.0
                a_stop = jnp.where(speed > 0.5, neg15, zero)
                a_follow = jnp.clip(0.3 * (lg - (1.5 * speed + 2.0)), -4.0, 2.0)
                a_yield = jnp.where(lt < 2.0, neg15, zero)
                ab = jnp.where(cvf == 1.0, a_stop, zero)
                ab = jnp.where(cvf == 2.0, a_follow, ab)
                ab = jnp.where(cvf == 3.0, a_yield, ab)
                ab = jnp.clip(ab, -4.0, 2.0)

                quad = (e_s * e_s * w0 + d1sq * w1 + e_v * e_v * w2
                        + ab * ab * w3)
                lp = -0.5 * (quad + ld + C4)
                fv = feas_v[pl.ds(ii * LANES, LANES)]
                out_v[pl.ds(ii * LANES, LANES)] = jnp.where(
                    fv > 0.5, lp, jnp.full((LANES,), -1e4, jnp.float32))

            finish(i0, pxa, pya, qxa, qya, vxa, vya, spda, bj0a, bj1a)
            finish(i0 + 1, pxb, pyb, qxb, qyb, vxb, vyb, spdb, bj0b, bj1b)
            return carry

        lax.fori_loop(0, PER_W // 2, block_body, 0)
        pltpu.sync_copy(out_v, out_hbm.at[pl.ds(wbase, PTS_W)])

    return sc_kernel


def kernel(x, ctx, feasible_actions, action_path_type, action_constraint_type,
           comparable_metrics, path_polyline_idx, map_polylines, w_by_family,
           sigma):
    B, N, T, _ = x.shape
    A = action_path_type.shape[0]
    _, M, L, _ = map_polylines.shape
    NT = N * T
    LP = 128

    pts = jnp.concatenate([
        ctx[..., 0].reshape(-1),
        ctx[..., 1].reshape(-1),
        x[..., 0].reshape(-1),
        x[..., 1].reshape(-1),
        ctx[..., 3].reshape(-1),
        ctx[..., 4].reshape(-1),
    ])
    aux = jnp.concatenate([
        comparable_metrics[..., 1].transpose(0, 3, 1, 2).reshape(-1),
        comparable_metrics[..., 2].transpose(0, 3, 1, 2).reshape(-1),
        feasible_actions.transpose(0, 3, 1, 2).reshape(-1).astype(jnp.float32),
    ])

    mx = map_polylines[..., 0].reshape(B * M, L)
    my = map_polylines[..., 1].reshape(B * M, L)
    mx = jnp.concatenate([mx, jnp.repeat(mx[:, -1:], LP - L, axis=1)], axis=1)
    my = jnp.concatenate([my, jnp.repeat(my[:, -1:], LP - L, axis=1)], axis=1)

    pidx = jnp.zeros((32,), jnp.int32).at[: B * A].set(
        path_polyline_idx.reshape(-1).astype(jnp.int32))

    w = w_by_family[action_path_type]
    var = (sigma ** 2)[None, :] / jnp.maximum(w, 1e-6)
    inv_var = 1.0 / jnp.maximum(var, 1e-12)
    log_det = jnp.log(jnp.maximum(var, 1e-12)).sum(-1)
    consts = (jnp.zeros((64,), jnp.float32)
              .at[: A * 4].set(inv_var.reshape(-1))
              .at[32 : 32 + A].set(log_det)
              .at[40 : 40 + A].set(action_constraint_type.astype(jnp.float32)))

    sc = _make_sc_kernel(B, A, NT, M, L, LP)
    out = sc(mx, my, pidx, pts, aux, consts)
    return out.reshape(B, A, N, T).transpose(0, 2, 3, 1)
